# baseline (speedup 1.0000x reference)
import functools

import jax
import jax.numpy as jnp
from jax import lax
from jax.experimental import pallas as pl
from jax.experimental.pallas import tpu as pltpu
import jax.experimental.pallas.tpu_sc as plsc

_N = 10000
_E = 160000
_NUM_ELEM = 90
_SPHERE_C = 128
_EDGE_C = 128
_NUM_GAUSS = 600
_CUTOFF = 5.0
_LMAX = 6
_NUM_COEF = (_LMAX + 1) ** 2
_M0 = _LMAX + 1
_AVG_DEGREE = 23.395238876342773

_NC, _NS = 2, 16
_W = _NC * _NS

_GPAD = 640
_EXT = 144


def _sc_mesh():
    return plsc.VectorSubcoreMesh(
        core_axis_name="c", subcore_axis_name="s",
        num_cores=_NC, num_subcores=_NS)


def _sc_gather_atoms(flat_idx, atom):
    tot = flat_idx.shape[0]
    ch = 2000
    n_chunks = tot // ch
    per_w = n_chunks // _W

    def body(flat_hbm, atom_hbm, out_hbm, atom_v, idx_v, out_v):
        wid = lax.axis_index("s") * _NC + lax.axis_index("c")
        pltpu.sync_copy(atom_hbm, atom_v)

        def chunk_body(i, carry):
            base = (wid * per_w + i) * ch
            pltpu.sync_copy(flat_hbm.at[pl.ds(base, ch)], idx_v)

            def vec_body(j, c2):
                ids = idx_v[pl.ds(j * 16, 16)]
                out_v[pl.ds(j * 16, 16)] = plsc.load_gather(atom_v, [ids])
                return c2
            lax.fori_loop(0, ch // 16, vec_body, 0)
            pltpu.sync_copy(out_v, out_hbm.at[pl.ds(base, ch)])
            return carry
        lax.fori_loop(0, per_w, chunk_body, 0)

    f = pl.kernel(
        body,
        out_type=jax.ShapeDtypeStruct((tot,), jnp.int32),
        mesh=_sc_mesh(),
        compiler_params=pltpu.CompilerParams(needs_layout_passes=False),
        scratch_types=[
            pltpu.VMEM((atom.shape[0],), jnp.int32),
            pltpu.VMEM((ch,), jnp.int32),
            pltpu.VMEM((ch,), jnp.int32),
        ])
    return f(flat_idx, atom)


def _tc_edge_mlp(dist3, asrc3, adst3, w1g, src_pre, dst_pre, w2, b2r,
                 goff, g):
    b = dist3.shape[2]
    delta = _CUTOFF / (_NUM_GAUSS - 1)
    coeff = -0.5 / (2.0 * delta) ** 2

    dnT = (((0,), (0,)), ((), ()))

    import math
    k = math.sqrt(-coeff * math.log2(math.e))

    def body(d_ref, s_ref, t_ref, w1_ref, sp_ref, dp_ref, w2_ref, b2_ref,
             o_ref):
        d = d_ref[0]
        gs = lax.broadcasted_iota(jnp.int32, (_GPAD, 1), 0).astype(
            jnp.float32)
        t = d * k - gs * (delta * k)
        featT = jnp.exp2(t * (-t))
        el = lax.broadcasted_iota(jnp.int32, (128, 1), 0)
        ohsT = jnp.where(s_ref[0, 0] == el, 1.0, 0.0)
        ohtT = jnp.where(t_ref[0, 0] == el, 1.0, 0.0)
        z = (lax.dot_general(ohsT, sp_ref[...], dnT,
                             preferred_element_type=jnp.float32)
             + lax.dot_general(ohtT, dp_ref[...], dnT,
                               preferred_element_type=jnp.float32))
        h1 = lax.dot_general(featT, w1_ref[...], dnT,
                             preferred_element_type=jnp.float32) + z
        h1 = h1 * jax.nn.sigmoid(h1)
        h2 = jnp.dot(h1, w2_ref[...],
                     preferred_element_type=jnp.float32) + b2_ref[...]
        h2 = h2 * jax.nn.sigmoid(h2)
        ext = jnp.concatenate(
            [h2, jnp.ones((b, 1), jnp.float32),
             jnp.zeros((b, _EXT - 129), jnp.float32)], axis=1)
        o_ref[...] = ext

    full = lambda i: (0, 0)
    return pl.pallas_call(
        body,
        grid=(g,),
        in_specs=[
            pl.BlockSpec((1, 1, b), lambda i: (i + goff, 0, 0)),
            pl.BlockSpec((1, 1, 1, b), lambda i: (0, i + goff, 0, 0)),
            pl.BlockSpec((1, 1, 1, b), lambda i: (1, i + goff, 0, 0)),
            pl.BlockSpec(w1g.shape, full),
            pl.BlockSpec(src_pre.shape, full),
            pl.BlockSpec(dst_pre.shape, full),
            pl.BlockSpec(w2.shape, full),
            pl.BlockSpec(b2r.shape, full),
        ],
        out_specs=pl.BlockSpec((b, _EXT), lambda i: (i, 0)),
        out_shape=jax.ShapeDtypeStruct((g * b, _EXT), jnp.float32),
    )(dist3, asrc3, adst3, w1g, src_pre, dst_pre, w2, b2r)


def _sc_segment_sum(h2ext, edge_index, e_off):
    e = h2ext.shape[0]
    per_s = (e // _W) & ~7
    tail = e - _W * per_s
    ch = 128
    assert 0 <= tail <= ch and tail % 8 == 0
    sizes = [ch] * (per_s // ch) + ([per_s % ch] if per_s % ch else [])
    rows_per_s = _N // _NS
    zb = 25

    def body(h_hbm, dst_hbm, out_hbm, acc, rows_v, idx_v, zero_v, sems,
             zsem):
        c = lax.axis_index("c")
        s = lax.axis_index("s")
        zv = jnp.zeros((16,), jnp.float32)

        def zrow(r, carry):
            for k in range(_EXT // 16):
                zero_v[r, pl.ds(k * 16, 16)] = zv
            return carry
        lax.fori_loop(0, zb, zrow, 0)
        zcps = [pltpu.async_copy(
            zero_v, acc.at[pl.ds(s * rows_per_s + i * zb, zb)], zsem)
            for i in range(rows_per_s // zb)]
        for cp in zcps:
            cp.wait()
        plsc.subcore_barrier()

        base_s = (s * _NC + c) * per_s

        def issue(i, buf):
            off = base_s + i * ch
            sz = sizes[i]
            cpi = pltpu.async_copy(dst_hbm.at[1, pl.ds(e_off + off, sz)],
                                   idx_v.at[buf, pl.ds(0, sz)],
                                   sems.at[buf])
            cpr = pltpu.async_copy(h_hbm.at[pl.ds(off, sz)],
                                   rows_v.at[buf, pl.ds(0, sz)],
                                   sems.at[buf])
            return cpi, cpr

        pending = issue(0, 0)
        for i in range(len(sizes)):
            nxt = issue(i + 1, (i + 1) % 2) if i + 1 < len(sizes) else None
            for cp in pending:
                cp.wait()
            sz = sizes[i]
            pltpu.sync_copy(rows_v.at[i % 2, pl.ds(0, sz)],
                            acc.at[idx_v.at[i % 2, pl.ds(0, sz)]],
                            add=True)
            pending = nxt
        if tail:
            @pl.when(jnp.logical_and(c == 0, s == 0))
            def _():
                toff = _W * per_s
                pltpu.sync_copy(dst_hbm.at[1, pl.ds(e_off + toff, tail)],
                                idx_v.at[0, pl.ds(0, tail)])
                pltpu.sync_copy(h_hbm.at[pl.ds(toff, tail)],
                                rows_v.at[0, pl.ds(0, tail)])
                pltpu.sync_copy(rows_v.at[0, pl.ds(0, tail)],
                                acc.at[idx_v.at[0, pl.ds(0, tail)]],
                                add=True)
        plsc.subcore_barrier()
        pltpu.sync_copy(acc.at[pl.ds(s * rows_per_s, rows_per_s)],
                        out_hbm.at[c, pl.ds(s * rows_per_s, rows_per_s)])

    f = pl.kernel(
        body,
        out_type=jax.ShapeDtypeStruct((_NC, _N, _EXT), jnp.float32),
        mesh=_sc_mesh(),
        compiler_params=pltpu.CompilerParams(use_tc_tiling_on_sc=False),
        scratch_types=[
            pltpu.VMEM_SHARED((_N, _EXT), jnp.float32),
            pltpu.VMEM((2, ch, _EXT), jnp.float32),
            pltpu.VMEM((2, ch), jnp.int32),
            pltpu.VMEM((zb, _EXT), jnp.float32),
            pltpu.SemaphoreType.DMA((2,)),
            pltpu.SemaphoreType.DMA,
        ])
    return f(h2ext, edge_index)


def _tc_finalize(s_a, s_b, atom3, sphere_pad, w3s, b3r):
    gn, bn = atom3.shape[0], atom3.shape[1]
    m0_idx = [l * l + l for l in range(_LMAX + 1)]

    def body(sa_ref, sb_ref, a_ref, sph_ref, w3_ref, b3_ref, o_ref):
        ssum = (sa_ref[0] + sa_ref[1]) + (sb_ref[0] + sb_ref[1])
        h = ssum[:, :_EDGE_C]
        deg = ssum[:, _EDGE_C:_EDGE_C + 1]
        agg = (jnp.dot(h, w3_ref[...], preferred_element_type=jnp.float32)
               + deg * b3_ref[...])
        el = lax.broadcasted_iota(jnp.int32, (1, 128), 1)
        oh = jnp.where(a_ref[0] == el, 1.0, 0.0)
        sph = jnp.dot(oh, sph_ref[...],
                      preferred_element_type=jnp.float32)
        o_ref[...] = jnp.zeros((bn, _NUM_COEF, _SPHERE_C), jnp.float32)
        for l, idx in enumerate(m0_idx):
            row = agg[:, l * _SPHERE_C:(l + 1) * _SPHERE_C]
            if idx == 0:
                row = row + sph
            o_ref[:, idx, :] = row

    full = lambda i: (0, 0)
    return pl.pallas_call(
        body,
        grid=(gn,),
        in_specs=[
            pl.BlockSpec((_NC, bn, _EXT), lambda i: (0, i, 0)),
            pl.BlockSpec((_NC, bn, _EXT), lambda i: (0, i, 0)),
            pl.BlockSpec((1, bn, 1), lambda i: (i, 0, 0)),
            pl.BlockSpec(sphere_pad.shape, full),
            pl.BlockSpec(w3s.shape, full),
            pl.BlockSpec(b3r.shape, full),
        ],
        out_specs=pl.BlockSpec((bn, _NUM_COEF, _SPHERE_C),
                               lambda i: (i, 0, 0)),
        out_shape=jax.ShapeDtypeStruct((_N, _NUM_COEF, _SPHERE_C),
                                       jnp.float32),
    )(s_a, s_b, atom3, sphere_pad, w3s, b3r)


def kernel(atomic_numbers, edge_index, edge_distance, sphere_table, src_table,
           dst_table, w1, b1, w2, b2, w3, b3):
    f32 = jnp.float32
    atomic_numbers = atomic_numbers.astype(jnp.int32)
    edge_index = edge_index.astype(jnp.int32)

    w1g = jnp.zeros((_GPAD, _EDGE_C), f32).at[:_NUM_GAUSS].set(
        w1[:_NUM_GAUSS])
    src_pre = jnp.zeros((128, _EDGE_C), f32).at[:_NUM_ELEM].set(
        src_table @ w1[_NUM_GAUSS:_NUM_GAUSS + _EDGE_C] + b1[None, :])
    dst_pre = jnp.zeros((128, _EDGE_C), f32).at[:_NUM_ELEM].set(
        dst_table @ w1[_NUM_GAUSS + _EDGE_C:])
    sphere_pad = jnp.zeros((128, _SPHERE_C), f32).at[:_NUM_ELEM].set(
        sphere_table)
    w3s = (w3 / _AVG_DEGREE).astype(f32)
    b3r = (b3 / _AVG_DEGREE)[None, :].astype(f32)
    b2r = b2[None, :].astype(f32)

    a_flat = _sc_gather_atoms(edge_index.reshape(-1), atomic_numbers)

    eb = 4000
    g = _E // eb
    gh = g // 2
    eh = _E // 2
    a2 = a_flat.reshape(2, g, 1, eb)
    dist3 = edge_distance.astype(f32).reshape(g, 1, eb)
    w2f = w2.astype(f32)
    h_a = _tc_edge_mlp(dist3, a2, a2, w1g, src_pre, dst_pre, w2f, b2r,
                       0, gh)
    s_a = _sc_segment_sum(h_a, edge_index, 0)
    h_b = _tc_edge_mlp(dist3, a2, a2, w1g, src_pre, dst_pre, w2f, b2r,
                       gh, gh)
    s_b = _sc_segment_sum(h_b, edge_index, eh)

    bn = 400
    gn = _N // bn
    return _tc_finalize(s_a, s_b, atomic_numbers.reshape(gn, bn, 1),
                        sphere_pad, w3s, b3r)


# speedup vs baseline: 1.3493x; 1.3493x over previous
import functools

import jax
import jax.numpy as jnp
from jax import lax
from jax.experimental import pallas as pl
from jax.experimental.pallas import tpu as pltpu
import jax.experimental.pallas.tpu_sc as plsc

_N = 10000
_E = 160000
_NUM_ELEM = 90
_SPHERE_C = 128
_EDGE_C = 128
_NUM_GAUSS = 600
_CUTOFF = 5.0
_LMAX = 6
_NUM_COEF = (_LMAX + 1) ** 2
_M0 = _LMAX + 1
_AVG_DEGREE = 23.395238876342773

_NC, _NS = 2, 16
_W = _NC * _NS

_GPAD = 640
_EXT = 144


def _sc_mesh():
    return plsc.VectorSubcoreMesh(
        core_axis_name="c", subcore_axis_name="s",
        num_cores=_NC, num_subcores=_NS)


def _sc_gather_atoms(flat_idx, atom):
    tot = flat_idx.shape[0]
    ch = 2000
    n_chunks = tot // ch
    per_w = n_chunks // _W

    def body(flat_hbm, atom_hbm, out_hbm, atom_v, idx_v, out_v):
        wid = lax.axis_index("s") * _NC + lax.axis_index("c")
        pltpu.sync_copy(atom_hbm, atom_v)

        def chunk_body(i, carry):
            base = (wid * per_w + i) * ch
            pltpu.sync_copy(flat_hbm.at[pl.ds(base, ch)], idx_v)

            def vec_body(j, c2):
                ids = idx_v[pl.ds(j * 16, 16)]
                out_v[pl.ds(j * 16, 16)] = plsc.load_gather(atom_v, [ids])
                return c2
            lax.fori_loop(0, ch // 16, vec_body, 0)
            pltpu.sync_copy(out_v, out_hbm.at[pl.ds(base, ch)])
            return carry
        lax.fori_loop(0, per_w, chunk_body, 0)

    f = pl.kernel(
        body,
        out_type=jax.ShapeDtypeStruct((tot,), jnp.int32),
        mesh=_sc_mesh(),
        compiler_params=pltpu.CompilerParams(needs_layout_passes=False),
        scratch_types=[
            pltpu.VMEM((atom.shape[0],), jnp.int32),
            pltpu.VMEM((ch,), jnp.int32),
            pltpu.VMEM((ch,), jnp.int32),
        ])
    return f(flat_idx, atom)


def _tc_edge_mlp(dist3, asrc3, adst3, w1g, src_pre, dst_pre, w2, b2r,
                 goff, g):
    b = dist3.shape[2]
    delta = _CUTOFF / (_NUM_GAUSS - 1)
    coeff = -0.5 / (2.0 * delta) ** 2

    dnT = (((0,), (0,)), ((), ()))

    import math
    k = math.sqrt(-coeff * math.log2(math.e))

    def body(d_ref, s_ref, t_ref, w1_ref, sp_ref, dp_ref, w2_ref, b2_ref,
             o_ref):
        d = d_ref[0]
        gs = lax.broadcasted_iota(jnp.int32, (_GPAD, 1), 0).astype(
            jnp.float32)
        t = d * k - gs * (delta * k)
        featT = jnp.exp2(t * (-t))
        el = lax.broadcasted_iota(jnp.int32, (128, 1), 0)
        ohsT = jnp.where(s_ref[0, 0] == el, 1.0, 0.0)
        ohtT = jnp.where(t_ref[0, 0] == el, 1.0, 0.0)
        z = (lax.dot_general(ohsT, sp_ref[...], dnT,
                             preferred_element_type=jnp.float32)
             + lax.dot_general(ohtT, dp_ref[...], dnT,
                               preferred_element_type=jnp.float32))
        h1 = lax.dot_general(featT, w1_ref[...], dnT,
                             preferred_element_type=jnp.float32) + z
        h1 = h1 * jax.nn.sigmoid(h1)
        h2 = jnp.dot(h1, w2_ref[...],
                     preferred_element_type=jnp.float32) + b2_ref[...]
        h2 = h2 * jax.nn.sigmoid(h2)
        o_ref[...] = h2

    full = lambda i: (0, 0)
    return pl.pallas_call(
        body,
        grid=(g,),
        in_specs=[
            pl.BlockSpec((1, 1, b), lambda i: (i + goff, 0, 0)),
            pl.BlockSpec((1, 1, 1, b), lambda i: (0, i + goff, 0, 0)),
            pl.BlockSpec((1, 1, 1, b), lambda i: (1, i + goff, 0, 0)),
            pl.BlockSpec(w1g.shape, full),
            pl.BlockSpec(src_pre.shape, full),
            pl.BlockSpec(dst_pre.shape, full),
            pl.BlockSpec(w2.shape, full),
            pl.BlockSpec(b2r.shape, full),
        ],
        out_specs=pl.BlockSpec((b, _EDGE_C), lambda i: (i, 0)),
        out_shape=jax.ShapeDtypeStruct((g * b, _EDGE_C), jnp.float32),
    )(dist3, asrc3, adst3, w1g, src_pre, dst_pre, w2, b2r)


def _sc_segment_sum(h2ext, edge_index, e_off):
    e = h2ext.shape[0]
    per_s = (e // _W) & ~7
    tail = e - _W * per_s
    ch = 128
    assert 0 <= tail <= ch and tail % 8 == 0
    sizes = [ch] * (per_s // ch) + ([per_s % ch] if per_s % ch else [])
    rows_per_s = _N // _NS
    zb = 25

    def body(h_hbm, dst_hbm, ones_hbm, out_hbm, dout_hbm, acc, dacc,
             rows_v, idx_v, zero_v, ones_v, sems, zsem):
        c = lax.axis_index("c")
        s = lax.axis_index("s")
        zv = jnp.zeros((16,), jnp.float32)

        def zrow(r, carry):
            for k in range(_EDGE_C // 16):
                zero_v[r, pl.ds(k * 16, 16)] = zv
            return carry
        lax.fori_loop(0, zb, zrow, 0)
        pltpu.sync_copy(ones_hbm, ones_v)
        zcps = [pltpu.async_copy(
            zero_v, acc.at[pl.ds(s * rows_per_s + i * zb, zb)], zsem)
            for i in range(rows_per_s // zb)]
        zcps += [pltpu.async_copy(
            zero_v.at[pl.ds(0, zb), pl.ds(0, 8)],
            dacc.at[pl.ds(s * rows_per_s + i * zb, zb)], zsem)
            for i in range(rows_per_s // zb)]
        for cp in zcps:
            cp.wait()
        plsc.subcore_barrier()

        base_s = (s * _NC + c) * per_s

        def issue(i, buf):
            off = base_s + i * ch
            sz = sizes[i]
            cpi = pltpu.async_copy(dst_hbm.at[1, pl.ds(e_off + off, sz)],
                                   idx_v.at[buf, pl.ds(0, sz)],
                                   sems.at[buf])
            cpr = pltpu.async_copy(h_hbm.at[pl.ds(off, sz)],
                                   rows_v.at[buf, pl.ds(0, sz)],
                                   sems.at[buf])
            return cpi, cpr

        pending = issue(0, 0)
        for i in range(len(sizes)):
            nxt = issue(i + 1, (i + 1) % 2) if i + 1 < len(sizes) else None
            for cp in pending:
                cp.wait()
            sz = sizes[i]
            pltpu.sync_copy(rows_v.at[i % 2, pl.ds(0, sz)],
                            acc.at[idx_v.at[i % 2, pl.ds(0, sz)]],
                            add=True)
            pltpu.sync_copy(ones_v.at[pl.ds(0, sz)],
                            dacc.at[idx_v.at[i % 2, pl.ds(0, sz)]],
                            add=True)
            pending = nxt
        if tail:
            @pl.when(jnp.logical_and(c == 0, s == 0))
            def _():
                toff = _W * per_s
                pltpu.sync_copy(dst_hbm.at[1, pl.ds(e_off + toff, tail)],
                                idx_v.at[0, pl.ds(0, tail)])
                pltpu.sync_copy(h_hbm.at[pl.ds(toff, tail)],
                                rows_v.at[0, pl.ds(0, tail)])
                pltpu.sync_copy(rows_v.at[0, pl.ds(0, tail)],
                                acc.at[idx_v.at[0, pl.ds(0, tail)]],
                                add=True)
                pltpu.sync_copy(ones_v.at[pl.ds(0, tail)],
                                dacc.at[idx_v.at[0, pl.ds(0, tail)]],
                                add=True)
        plsc.subcore_barrier()
        pltpu.sync_copy(acc.at[pl.ds(s * rows_per_s, rows_per_s)],
                        out_hbm.at[c, pl.ds(s * rows_per_s, rows_per_s)])
        pltpu.sync_copy(dacc.at[pl.ds(s * rows_per_s, rows_per_s)],
                        dout_hbm.at[c, pl.ds(s * rows_per_s, rows_per_s)])

    f = pl.kernel(
        body,
        out_type=[jax.ShapeDtypeStruct((_NC, _N, _EDGE_C), jnp.float32),
                  jax.ShapeDtypeStruct((_NC, _N, 8), jnp.float32)],
        mesh=_sc_mesh(),
        compiler_params=pltpu.CompilerParams(use_tc_tiling_on_sc=False),
        scratch_types=[
            pltpu.VMEM_SHARED((_N, _EDGE_C), jnp.float32),
            pltpu.VMEM_SHARED((_N, 8), jnp.float32),
            pltpu.VMEM((2, ch, _EDGE_C), jnp.float32),
            pltpu.VMEM((2, ch), jnp.int32),
            pltpu.VMEM((zb, _EDGE_C), jnp.float32),
            pltpu.VMEM((ch, 8), jnp.float32),
            pltpu.SemaphoreType.DMA((2,)),
            pltpu.SemaphoreType.DMA,
        ])
    return f(h2ext, edge_index, jnp.ones((ch, 8), jnp.float32))


def _tc_finalize(s_a, s_b, dg_a, dg_b, atom3, sphere_pad, w3s, b3r):
    gn, bn = atom3.shape[0], atom3.shape[1]
    m0_idx = [l * l + l for l in range(_LMAX + 1)]

    def body(sa_ref, sb_ref, da_ref, db_ref, a_ref, sph_ref, w3_ref,
             b3_ref, o_ref):
        h = (sa_ref[0] + sa_ref[1]) + (sb_ref[0] + sb_ref[1])
        dsum = (da_ref[0] + da_ref[1]) + (db_ref[0] + db_ref[1])
        deg = dsum[:, 0:1]
        agg = (jnp.dot(h, w3_ref[...], preferred_element_type=jnp.float32)
               + deg * b3_ref[...])
        el = lax.broadcasted_iota(jnp.int32, (1, 128), 1)
        oh = jnp.where(a_ref[0] == el, 1.0, 0.0)
        sph = jnp.dot(oh, sph_ref[...],
                      preferred_element_type=jnp.float32)
        o_ref[...] = jnp.zeros((bn, _NUM_COEF, _SPHERE_C), jnp.float32)
        for l, idx in enumerate(m0_idx):
            row = agg[:, l * _SPHERE_C:(l + 1) * _SPHERE_C]
            if idx == 0:
                row = row + sph
            o_ref[:, idx, :] = row

    full = lambda i: (0, 0)
    return pl.pallas_call(
        body,
        grid=(gn,),
        in_specs=[
            pl.BlockSpec((_NC, bn, _EDGE_C), lambda i: (0, i, 0)),
            pl.BlockSpec((_NC, bn, _EDGE_C), lambda i: (0, i, 0)),
            pl.BlockSpec((_NC, bn, 8), lambda i: (0, i, 0)),
            pl.BlockSpec((_NC, bn, 8), lambda i: (0, i, 0)),
            pl.BlockSpec((1, bn, 1), lambda i: (i, 0, 0)),
            pl.BlockSpec(sphere_pad.shape, full),
            pl.BlockSpec(w3s.shape, full),
            pl.BlockSpec(b3r.shape, full),
        ],
        out_specs=pl.BlockSpec((bn, _NUM_COEF, _SPHERE_C),
                               lambda i: (i, 0, 0)),
        out_shape=jax.ShapeDtypeStruct((_N, _NUM_COEF, _SPHERE_C),
                                       jnp.float32),
    )(s_a, s_b, dg_a, dg_b, atom3, sphere_pad, w3s, b3r)


def kernel(atomic_numbers, edge_index, edge_distance, sphere_table, src_table,
           dst_table, w1, b1, w2, b2, w3, b3):
    f32 = jnp.float32
    atomic_numbers = atomic_numbers.astype(jnp.int32)
    edge_index = edge_index.astype(jnp.int32)

    w1g = jnp.zeros((_GPAD, _EDGE_C), f32).at[:_NUM_GAUSS].set(
        w1[:_NUM_GAUSS])
    src_pre = jnp.zeros((128, _EDGE_C), f32).at[:_NUM_ELEM].set(
        src_table @ w1[_NUM_GAUSS:_NUM_GAUSS + _EDGE_C] + b1[None, :])
    dst_pre = jnp.zeros((128, _EDGE_C), f32).at[:_NUM_ELEM].set(
        dst_table @ w1[_NUM_GAUSS + _EDGE_C:])
    sphere_pad = jnp.zeros((128, _SPHERE_C), f32).at[:_NUM_ELEM].set(
        sphere_table)
    w3s = (w3 / _AVG_DEGREE).astype(f32)
    b3r = (b3 / _AVG_DEGREE)[None, :].astype(f32)
    b2r = b2[None, :].astype(f32)

    a_flat = _sc_gather_atoms(edge_index.reshape(-1), atomic_numbers)

    eb = 4000
    g = _E // eb
    gh = g // 2
    eh = _E // 2
    a2 = a_flat.reshape(2, g, 1, eb)
    dist3 = edge_distance.astype(f32).reshape(g, 1, eb)
    w2f = w2.astype(f32)
    h_a = _tc_edge_mlp(dist3, a2, a2, w1g, src_pre, dst_pre, w2f, b2r,
                       0, gh)
    s_a, dg_a = _sc_segment_sum(h_a, edge_index, 0)
    h_b = _tc_edge_mlp(dist3, a2, a2, w1g, src_pre, dst_pre, w2f, b2r,
                       gh, gh)
    s_b, dg_b = _sc_segment_sum(h_b, edge_index, eh)

    bn = 400
    gn = _N // bn
    return _tc_finalize(s_a, s_b, dg_a, dg_b,
                        atomic_numbers.reshape(gn, bn, 1),
                        sphere_pad, w3s, b3r)
